# Initial kernel scaffold; baseline (speedup 1.0000x reference)
import jax
import jax.numpy as jnp
from jax.experimental import pallas as pl


def kernel(class_logits, box_regression, proposals):
    raise NotImplementedError("write your pallas kernel here")


# speedup vs baseline: 14.3953x; 14.3953x over previous
import math

import jax
import jax.numpy as jnp
from jax.experimental import pallas as pl

_IMG = 800.0
_NCLS = 21
_NPROP = 5000
_PRE = 1000
_PREP = 1024
_DET = 100
_DETP = 128
_SCORE_TH = 0.05
_NMS_TH = 0.5
_CLIP = math.log(1000.0 / 16.0)
_NEG = -1e9


def _prep_kernel(logits_ref, dx_ref, dy_ref, dw_ref, dh_ref, prop_ref,
                 x1_ref, y1_ref, x2_ref, y2_ref, ms_ref):
    logits = logits_ref[...]
    m = jnp.max(logits, axis=1, keepdims=True)
    e = jnp.exp(logits - m)
    s = jnp.sum(e, axis=1, keepdims=True)
    probs = (e / s)[:, 1:]
    ms_ref[...] = jnp.where(probs > _SCORE_TH, probs, _NEG)

    p = prop_ref[...]
    w = p[:, 2:3] - p[:, 0:1]
    h = p[:, 3:4] - p[:, 1:2]
    cx = p[:, 0:1] + 0.5 * w
    cy = p[:, 1:2] + 0.5 * h
    pcx = dx_ref[...] * w + cx
    pcy = dy_ref[...] * h + cy
    pw = jnp.exp(jnp.minimum(dw_ref[...], _CLIP)) * w
    ph = jnp.exp(jnp.minimum(dh_ref[...], _CLIP)) * h
    x1_ref[...] = jnp.clip(pcx - 0.5 * pw, 0.0, _IMG)
    y1_ref[...] = jnp.clip(pcy - 0.5 * ph, 0.0, _IMG)
    x2_ref[...] = jnp.clip(pcx + 0.5 * pw, 0.0, _IMG)
    y2_ref[...] = jnp.clip(pcy + 0.5 * ph, 0.0, _IMG)


def _nms_kernel(feat_ref, featT_ref, out_ref):
    off_c = feat_ref[:, 5:6] * (_IMG + 2.0)
    x1c = feat_ref[:, 0:1] + off_c
    y1c = feat_ref[:, 1:2] + off_c
    x2c = feat_ref[:, 2:3] + off_c
    y2c = feat_ref[:, 3:4] + off_c
    off_r = featT_ref[5:6, :] * (_IMG + 2.0)
    x1r = featT_ref[0:1, :] + off_r
    y1r = featT_ref[1:2, :] + off_r
    x2r = featT_ref[2:3, :] + off_r
    y2r = featT_ref[3:4, :] + off_r

    area_c = (x2c - x1c) * (y2c - y1c)
    area_r = (x2r - x1r) * (y2r - y1r)
    iw = jnp.clip(jnp.minimum(x2c, x2r) - jnp.maximum(x1c, x1r), 0.0, None)
    ih = jnp.clip(jnp.minimum(y2c, y2r) - jnp.maximum(y1c, y1r), 0.0, None)
    inter = iw * ih
    overl = inter > _NMS_TH * (area_c + area_r - inter + 1e-9)

    r_io = jax.lax.broadcasted_iota(jnp.int32, (_PREP, _PREP), 0)
    c_io = jax.lax.broadcasted_iota(jnp.int32, (_PREP, _PREP), 1)
    upper = r_io < c_io
    t_mat = jnp.where(overl & upper, 1.0, 0.0)
    su = jnp.where(upper, 1.0, 0.0)

    valid = jnp.where(featT_ref[4:5, :] > 0.0, 1.0, 0.0)

    def cond(state):
        _, changed = state
        return changed

    def body(state):
        keep, _ = state
        sup = jnp.dot(keep, t_mat, preferred_element_type=jnp.float32)
        new = jnp.where(sup > 0.5, 0.0, valid)
        return new, jnp.any(new != keep)

    keep, _ = jax.lax.while_loop(cond, body, (valid, jnp.bool_(True)))

    pk = jnp.dot(keep, su, preferred_element_type=jnp.float32)
    iot = jax.lax.broadcasted_iota(jnp.int32, (1, _PREP), 1).astype(jnp.float32)
    total = jnp.sum(keep)
    rank = jnp.where(keep > 0.5, pk, total + (iot - pk))
    pp = jax.lax.broadcasted_iota(jnp.int32, (_DETP, 1), 0).astype(jnp.float32)
    onehot = jnp.where(rank == pp, 1.0, 0.0)
    g = jax.lax.dot(onehot, feat_ref[...],
                    precision=jax.lax.Precision.HIGHEST,
                    preferred_element_type=jnp.float32)
    ok = jnp.where(pp < total, 1.0, 0.0)
    out_ref[...] = g * ok


def kernel(class_logits, box_regression, proposals):
    r = box_regression.reshape(_NPROP, _NCLS, 4)[:, 1:, :]
    dx, dy, dw, dh = r[..., 0], r[..., 1], r[..., 2], r[..., 3]

    x1, y1, x2, y2, masked = pl.pallas_call(
        _prep_kernel,
        out_shape=[jax.ShapeDtypeStruct((_NPROP, _NCLS - 1), jnp.float32)] * 5,
    )(class_logits, dx, dy, dw, dh, proposals)

    masked_f = masked.reshape(-1)
    top_vals, top_idx = jax.lax.top_k(masked_f, _PRE)
    labels = (top_idx % (_NCLS - 1) + 1).astype(jnp.float32)
    cols = [x1.reshape(-1)[top_idx], y1.reshape(-1)[top_idx],
            x2.reshape(-1)[top_idx], y2.reshape(-1)[top_idx],
            top_vals, labels]
    feat = jnp.stack(cols, axis=1)
    pad_rows = jnp.concatenate(
        [jnp.zeros((_PREP - _PRE, 4), jnp.float32),
         jnp.full((_PREP - _PRE, 1), _NEG, jnp.float32),
         jnp.zeros((_PREP - _PRE, 1), jnp.float32)], axis=1)
    feat = jnp.concatenate([feat, pad_rows], axis=0)
    feat = jnp.concatenate([feat, jnp.zeros((_PREP, 2), jnp.float32)], axis=1)

    g = pl.pallas_call(
        _nms_kernel,
        out_shape=jax.ShapeDtypeStruct((_DETP, 8), jnp.float32),
    )(feat, feat.T)
    return g[:_DET, :6]
